# Initial kernel scaffold; baseline (speedup 1.0000x reference)
import jax
import jax.numpy as jnp
from jax.experimental import pallas as pl


def kernel(hour_idx, x1, x2, b0, b1, b2):
    raise NotImplementedError("write your pallas kernel here")


# speedup vs baseline: 1.1213x; 1.1213x over previous
import jax
import jax.numpy as jnp
from jax.experimental import pallas as pl
from jax.experimental.pallas import tpu as pltpu

_ROWS = 16384
_COLS = 1024
_BLK = 1024


def _body(idx_ref, b0_ref, b1_ref, b2_ref, x1_ref, x2_ref, o_ref):
    h = idx_ref[0]
    c0 = b0_ref[h]
    c1 = b1_ref[h]
    c2 = b2_ref[h]
    o_ref[:] = c0 + c1 * x1_ref[:] + c2 * x2_ref[:]


def kernel(hour_idx, x1, x2, b0, b1, b2):
    idx = jnp.asarray(hour_idx, jnp.int32).reshape(1)
    grid = (_ROWS // _BLK,)
    out = pl.pallas_call(
        _body,
        grid=grid,
        in_specs=[
            pl.BlockSpec(memory_space=pltpu.SMEM),
            pl.BlockSpec(memory_space=pltpu.SMEM),
            pl.BlockSpec(memory_space=pltpu.SMEM),
            pl.BlockSpec(memory_space=pltpu.SMEM),
            pl.BlockSpec((_BLK, _COLS), lambda i: (i, 0)),
            pl.BlockSpec((_BLK, _COLS), lambda i: (i, 0)),
        ],
        out_specs=pl.BlockSpec((_BLK, _COLS), lambda i: (i, 0)),
        out_shape=jax.ShapeDtypeStruct((_ROWS, _COLS), jnp.float32),
    )(idx, b0, b1, b2, x1, x2)
    return out
